# Initial kernel scaffold; baseline (speedup 1.0000x reference)
import functools

import jax
import jax.numpy as jnp
from jax.experimental import pallas as pl
from jax.experimental.pallas import tpu as pltpu

_PAD = 128

_CP = pltpu.CompilerParams(
    dimension_semantics=("parallel",),
    vmem_limit_bytes=60 * 1024 * 1024,
)


def _taps(w):
    co, ci, kh, kw = w.shape
    return jnp.transpose(w, (2, 3, 0, 1)).reshape(kh * kw, co, ci)


def _flat(x):
    b, c, h, w = x.shape
    f = jnp.transpose(x, (1, 0, 2, 3)).reshape(c, b * h * w)
    return jnp.pad(f, ((0, 0), (_PAD, _PAD)))


def _tap_slice(x_ref, base, tm, dh, dw, hrow, wcol, h_, w_):
    sl = x_ref[:, pl.ds(base + dh * w_ + dw, tm)]
    cond = None
    if dh < 0:
        cond = hrow > 0
    elif dh > 0:
        cond = hrow < h_ - 1
    if dw < 0:
        c2 = wcol > 0
        cond = c2 if cond is None else (cond & c2)
    elif dw > 0:
        c2 = wcol < w_ - 1
        cond = c2 if cond is None else (cond & c2)
    if cond is not None:
        sl = jnp.where(cond, sl, jnp.zeros_like(sl))
    return sl


def _hi_kernel(wt_ref, x_ref, u_ref, y_ref, s_ref, q_ref, *, tm, h_, w_):
    m0 = pl.program_id(0) * tm
    idx = jax.lax.broadcasted_iota(jnp.int32, (1, tm), 1) + m0
    wcol = idx & (w_ - 1)
    hrow = (idx // w_) & (h_ - 1)
    acc = u_ref[...].astype(jnp.float32)
    for t in range(9):
        dh, dw = t // 3 - 1, t % 3 - 1
        sl = _tap_slice(x_ref, _PAD + m0, tm, dh, dw, hrow, wcol, h_, w_)
        acc = acc + jnp.dot(wt_ref[t], sl, preferred_element_type=jnp.float32)
    y_ref[...] = acc.astype(y_ref.dtype)
    s_ref[...] = jnp.sum(acc, axis=1, keepdims=True)[None]
    q_ref[...] = jnp.sum(acc * acc, axis=1, keepdims=True)[None]


def _lo_kernel(wh2l_ref, wl2l_ref, wl2h_ref, p_ref, x_ref,
               ylo_ref, yl2h_ref, s_ref, q_ref, *, tm, h_, w_):
    m0 = pl.program_id(0) * tm
    idx = jax.lax.broadcasted_iota(jnp.int32, (1, tm), 1) + m0
    wcol = idx & (w_ - 1)
    hrow = (idx // w_) & (h_ - 1)
    acc = jnp.zeros(ylo_ref.shape, jnp.float32)
    acc2 = jnp.zeros(yl2h_ref.shape, jnp.float32)
    for t in range(9):
        dh, dw = t // 3 - 1, t % 3 - 1
        sp = _tap_slice(p_ref, _PAD + m0, tm, dh, dw, hrow, wcol, h_, w_)
        sx = _tap_slice(x_ref, _PAD + m0, tm, dh, dw, hrow, wcol, h_, w_)
        acc = (acc + jnp.dot(wh2l_ref[t], sp, preferred_element_type=jnp.float32)
               + jnp.dot(wl2l_ref[t], sx, preferred_element_type=jnp.float32))
        acc2 = acc2 + jnp.dot(wl2h_ref[t], sx, preferred_element_type=jnp.float32)
    ylo_ref[...] = acc.astype(ylo_ref.dtype)
    yl2h_ref[...] = acc2.astype(yl2h_ref.dtype)
    s_ref[...] = jnp.sum(acc, axis=1, keepdims=True)[None]
    q_ref[...] = jnp.sum(acc * acc, axis=1, keepdims=True)[None]


def _affine_kernel(y_ref, scale_ref, shift_ref, o_ref):
    o_ref[...] = (y_ref[...].astype(jnp.float32) * scale_ref[...]
                  + shift_ref[...]).astype(o_ref.dtype)


def _bn_apply(y, part_s, part_q, m, gamma, beta, eps, tm):
    c, mp = y.shape
    mean = jnp.sum(part_s, axis=0)[:, 0] / m
    var = jnp.sum(part_q, axis=0)[:, 0] / m - mean * mean
    scale = gamma.astype(jnp.float32) * jax.lax.rsqrt(var + eps)
    shift = beta.astype(jnp.float32) - mean * scale
    n = mp // tm
    return pl.pallas_call(
        _affine_kernel,
        out_shape=jax.ShapeDtypeStruct((c, mp), jnp.float32),
        grid=(n,),
        in_specs=[
            pl.BlockSpec((c, tm), lambda i: (0, i)),
            pl.BlockSpec((c, 1), lambda i: (0, 0)),
            pl.BlockSpec((c, 1), lambda i: (0, 0)),
        ],
        out_specs=pl.BlockSpec((c, tm), lambda i: (0, i)),
        compiler_params=_CP,
    )(y, scale[:, None], shift[:, None])


def kernel(w_h2h, w_h2l, w_l2h, w_l2l, gamma_h, beta_h, gamma_l, beta_l,
           x_h, x_l, eps=1e-5):
    b, cih, h, w = x_h.shape
    _, cil, hl, wl = x_l.shape
    coh = w_h2h.shape[0]
    col = w_l2l.shape[0]
    assert h & (h - 1) == 0 and w & (w - 1) == 0, "spatial dims must be pow2"
    mh, ml = b * h * w, b * hl * wl

    xh_f = _flat(x_h)
    pool = x_h.reshape(b, cih, hl, 2, wl, 2).mean(axis=(3, 5))
    ph_f = _flat(pool)
    xl_f = _flat(x_l)

    tml = 512
    nl = ml // tml
    kl = functools.partial(_lo_kernel, tm=tml, h_=hl, w_=wl)
    mlp2 = ml + 2 * _PAD
    y_lo, y_l2h, s_lo, q_lo = pl.pallas_call(
        kl,
        out_shape=(jax.ShapeDtypeStruct((col, ml), jnp.float32),
                   jax.ShapeDtypeStruct((coh, ml), jnp.float32),
                   jax.ShapeDtypeStruct((nl, col, 1), jnp.float32),
                   jax.ShapeDtypeStruct((nl, col, 1), jnp.float32)),
        grid=(nl,),
        in_specs=[
            pl.BlockSpec((9, col, cih), lambda i: (0, 0, 0)),
            pl.BlockSpec((9, col, cil), lambda i: (0, 0, 0)),
            pl.BlockSpec((9, coh, cil), lambda i: (0, 0, 0)),
            pl.BlockSpec((cih, mlp2), lambda i: (0, 0)),
            pl.BlockSpec((cil, mlp2), lambda i: (0, 0)),
        ],
        out_specs=(pl.BlockSpec((col, tml), lambda i: (0, i)),
                   pl.BlockSpec((coh, tml), lambda i: (0, i)),
                   pl.BlockSpec((1, col, 1), lambda i: (i, 0, 0)),
                   pl.BlockSpec((1, col, 1), lambda i: (i, 0, 0))),
        compiler_params=_CP,
    )(_taps(w_h2l), _taps(w_l2l), _taps(w_l2h), ph_f, xl_f)

    u = jnp.repeat(jnp.repeat(y_l2h.reshape(coh, b, hl, wl), 2, axis=2),
                   2, axis=3).reshape(coh, mh)

    tmh = 512
    nh = mh // tmh
    kh = functools.partial(_hi_kernel, tm=tmh, h_=h, w_=w)
    mhp2 = mh + 2 * _PAD
    y_hi, s_hi, q_hi = pl.pallas_call(
        kh,
        out_shape=(jax.ShapeDtypeStruct((coh, mh), jnp.float32),
                   jax.ShapeDtypeStruct((nh, coh, 1), jnp.float32),
                   jax.ShapeDtypeStruct((nh, coh, 1), jnp.float32)),
        grid=(nh,),
        in_specs=[
            pl.BlockSpec((9, coh, cih), lambda i: (0, 0, 0)),
            pl.BlockSpec((cih, mhp2), lambda i: (0, 0)),
            pl.BlockSpec((coh, tmh), lambda i: (0, i)),
        ],
        out_specs=(pl.BlockSpec((coh, tmh), lambda i: (0, i)),
                   pl.BlockSpec((1, coh, 1), lambda i: (i, 0, 0)),
                   pl.BlockSpec((1, coh, 1), lambda i: (i, 0, 0))),
        compiler_params=_CP,
    )(_taps(w_h2h), xh_f, u)

    out_h2 = _bn_apply(y_hi, s_hi, q_hi, mh, gamma_h, beta_h, eps, tmh)
    out_l2 = _bn_apply(y_lo, s_lo, q_lo, ml, gamma_l, beta_l, eps, tml)

    out_h = jnp.transpose(out_h2.reshape(coh, b, h, w), (1, 0, 2, 3))
    out_l = jnp.transpose(out_l2.reshape(col, b, hl, wl), (1, 0, 2, 3))
    return out_h, out_l


# speedup vs baseline: 2.4802x; 2.4802x over previous
import functools

import jax
import jax.numpy as jnp
from jax.experimental import pallas as pl
from jax.experimental.pallas import tpu as pltpu

_PAD = 128

_CP = pltpu.CompilerParams(
    dimension_semantics=("parallel",),
    vmem_limit_bytes=60 * 1024 * 1024,
)


def _taps(w):
    co, ci, kh, kw = w.shape
    return jnp.transpose(w, (2, 3, 0, 1)).reshape(kh * kw, co, ci)


def _flat(x):
    b, c, h, w = x.shape
    f = jnp.transpose(x, (1, 0, 2, 3)).reshape(c, b * h * w)
    return jnp.pad(f, ((0, 0), (_PAD, _PAD)))


def _tap_slice(wide, tm, dh, dw, hrow, wcol, h_, w_):
    d = _PAD + dh * w_ + dw
    sl = jax.lax.slice_in_dim(wide, d, d + tm, axis=1)
    cond = None
    if dh < 0:
        cond = hrow > 0
    elif dh > 0:
        cond = hrow < h_ - 1
    if dw < 0:
        c2 = wcol > 0
        cond = c2 if cond is None else (cond & c2)
    elif dw > 0:
        c2 = wcol < w_ - 1
        cond = c2 if cond is None else (cond & c2)
    if cond is not None:
        sl = jnp.where(cond, sl, jnp.zeros_like(sl))
    return sl


def _hi_kernel(wt_ref, x_ref, u_ref, y_ref, s_ref, q_ref, *, tm, h_, w_):
    m0 = pl.program_id(0) * tm
    idx = jax.lax.broadcasted_iota(jnp.int32, (1, tm), 1) + m0
    wcol = idx & (w_ - 1)
    hrow = (idx // w_) & (h_ - 1)
    acc = u_ref[...].astype(jnp.float32)
    wide = x_ref[:, pl.ds(m0, tm + 2 * _PAD)]
    for t in range(9):
        dh, dw = t // 3 - 1, t % 3 - 1
        sl = _tap_slice(wide, tm, dh, dw, hrow, wcol, h_, w_)
        acc = acc + jnp.dot(wt_ref[t], sl, preferred_element_type=jnp.float32)
    y_ref[...] = acc.astype(y_ref.dtype)
    s_ref[...] = jnp.sum(acc, axis=1, keepdims=True)[None]
    q_ref[...] = jnp.sum(acc * acc, axis=1, keepdims=True)[None]


def _lo_kernel(wh2l_ref, wl2l_ref, wl2h_ref, p_ref, x_ref,
               ylo_ref, yl2h_ref, s_ref, q_ref, *, tm, h_, w_):
    m0 = pl.program_id(0) * tm
    idx = jax.lax.broadcasted_iota(jnp.int32, (1, tm), 1) + m0
    wcol = idx & (w_ - 1)
    hrow = (idx // w_) & (h_ - 1)
    acc = jnp.zeros(ylo_ref.shape, jnp.float32)
    acc2 = jnp.zeros(yl2h_ref.shape, jnp.float32)
    wide_p = p_ref[:, pl.ds(m0, tm + 2 * _PAD)]
    wide_x = x_ref[:, pl.ds(m0, tm + 2 * _PAD)]
    for t in range(9):
        dh, dw = t // 3 - 1, t % 3 - 1
        sp = _tap_slice(wide_p, tm, dh, dw, hrow, wcol, h_, w_)
        sx = _tap_slice(wide_x, tm, dh, dw, hrow, wcol, h_, w_)
        acc = (acc + jnp.dot(wh2l_ref[t], sp, preferred_element_type=jnp.float32)
               + jnp.dot(wl2l_ref[t], sx, preferred_element_type=jnp.float32))
        acc2 = acc2 + jnp.dot(wl2h_ref[t], sx, preferred_element_type=jnp.float32)
    ylo_ref[...] = acc.astype(ylo_ref.dtype)
    yl2h_ref[...] = acc2.astype(yl2h_ref.dtype)
    s_ref[...] = jnp.sum(acc, axis=1, keepdims=True)[None]
    q_ref[...] = jnp.sum(acc * acc, axis=1, keepdims=True)[None]


def _affine_kernel(y_ref, scale_ref, shift_ref, o_ref):
    o_ref[...] = (y_ref[...].astype(jnp.float32) * scale_ref[...]
                  + shift_ref[...]).astype(o_ref.dtype)


def _bn_apply(y, part_s, part_q, m, gamma, beta, eps, tm):
    c, mp = y.shape
    mean = jnp.sum(part_s, axis=0)[:, 0] / m
    var = jnp.sum(part_q, axis=0)[:, 0] / m - mean * mean
    scale = gamma.astype(jnp.float32) * jax.lax.rsqrt(var + eps)
    shift = beta.astype(jnp.float32) - mean * scale
    n = mp // tm
    return pl.pallas_call(
        _affine_kernel,
        out_shape=jax.ShapeDtypeStruct((c, mp), jnp.float32),
        grid=(n,),
        in_specs=[
            pl.BlockSpec((c, tm), lambda i: (0, i)),
            pl.BlockSpec((c, 1), lambda i: (0, 0)),
            pl.BlockSpec((c, 1), lambda i: (0, 0)),
        ],
        out_specs=pl.BlockSpec((c, tm), lambda i: (0, i)),
        compiler_params=_CP,
    )(y, scale[:, None], shift[:, None])


def kernel(w_h2h, w_h2l, w_l2h, w_l2l, gamma_h, beta_h, gamma_l, beta_l,
           x_h, x_l, eps=1e-5):
    b, cih, h, w = x_h.shape
    _, cil, hl, wl = x_l.shape
    coh = w_h2h.shape[0]
    col = w_l2l.shape[0]
    assert h & (h - 1) == 0 and w & (w - 1) == 0, "spatial dims must be pow2"
    mh, ml = b * h * w, b * hl * wl

    xh_f = _flat(x_h)
    pool = x_h.reshape(b, cih, hl, 2, wl, 2).mean(axis=(3, 5))
    ph_f = _flat(pool)
    xl_f = _flat(x_l)

    tml = min(512, ml)
    nl = ml // tml
    kl = functools.partial(_lo_kernel, tm=tml, h_=hl, w_=wl)
    mlp2 = ml + 2 * _PAD
    y_lo, y_l2h, s_lo, q_lo = pl.pallas_call(
        kl,
        out_shape=(jax.ShapeDtypeStruct((col, ml), jnp.float32),
                   jax.ShapeDtypeStruct((coh, ml), jnp.float32),
                   jax.ShapeDtypeStruct((nl, col, 1), jnp.float32),
                   jax.ShapeDtypeStruct((nl, col, 1), jnp.float32)),
        grid=(nl,),
        in_specs=[
            pl.BlockSpec((9, col, cih), lambda i: (0, 0, 0)),
            pl.BlockSpec((9, col, cil), lambda i: (0, 0, 0)),
            pl.BlockSpec((9, coh, cil), lambda i: (0, 0, 0)),
            pl.BlockSpec((cih, mlp2), lambda i: (0, 0)),
            pl.BlockSpec((cil, mlp2), lambda i: (0, 0)),
        ],
        out_specs=(pl.BlockSpec((col, tml), lambda i: (0, i)),
                   pl.BlockSpec((coh, tml), lambda i: (0, i)),
                   pl.BlockSpec((1, col, 1), lambda i: (i, 0, 0)),
                   pl.BlockSpec((1, col, 1), lambda i: (i, 0, 0))),
        compiler_params=_CP,
    )(_taps(w_h2l), _taps(w_l2l), _taps(w_l2h), ph_f, xl_f)

    u = jnp.repeat(jnp.repeat(y_l2h.reshape(coh, b, hl, wl), 2, axis=2),
                   2, axis=3).reshape(coh, mh)

    tmh = min(512, mh)
    nh = mh // tmh
    kh = functools.partial(_hi_kernel, tm=tmh, h_=h, w_=w)
    mhp2 = mh + 2 * _PAD
    y_hi, s_hi, q_hi = pl.pallas_call(
        kh,
        out_shape=(jax.ShapeDtypeStruct((coh, mh), jnp.float32),
                   jax.ShapeDtypeStruct((nh, coh, 1), jnp.float32),
                   jax.ShapeDtypeStruct((nh, coh, 1), jnp.float32)),
        grid=(nh,),
        in_specs=[
            pl.BlockSpec((9, coh, cih), lambda i: (0, 0, 0)),
            pl.BlockSpec((cih, mhp2), lambda i: (0, 0)),
            pl.BlockSpec((coh, tmh), lambda i: (0, i)),
        ],
        out_specs=(pl.BlockSpec((coh, tmh), lambda i: (0, i)),
                   pl.BlockSpec((1, coh, 1), lambda i: (i, 0, 0)),
                   pl.BlockSpec((1, coh, 1), lambda i: (i, 0, 0))),
        compiler_params=_CP,
    )(_taps(w_h2h), xh_f, u)

    out_h2 = _bn_apply(y_hi, s_hi, q_hi, mh, gamma_h, beta_h, eps, tmh)
    out_l2 = _bn_apply(y_lo, s_lo, q_lo, ml, gamma_l, beta_l, eps, tml)

    out_h = jnp.transpose(out_h2.reshape(coh, b, h, w), (1, 0, 2, 3))
    out_l = jnp.transpose(out_l2.reshape(col, b, hl, wl), (1, 0, 2, 3))
    return out_h, out_l
